# baseline (speedup 1.0000x reference)
import functools

import jax
import jax.numpy as jnp
from jax import lax
from jax.experimental import pallas as pl
from jax.experimental.pallas import tpu as pltpu
from jax.experimental.pallas import tpu_sc as plsc


@functools.lru_cache(maxsize=None)
def _make_gather(n_idx: int, vocab: int, emb: int):
    info = plsc.get_sparse_core_info()
    nc, ns = info.num_cores, info.num_subcores
    nw = nc * ns
    assert n_idx % (8 * nw) == 0
    per_w = n_idx // nw
    mesh = plsc.VectorSubcoreMesh(core_axis_name="c", subcore_axis_name="s")

    @functools.partial(
        pl.kernel,
        mesh=mesh,
        out_type=jax.ShapeDtypeStruct((n_idx, emb), jnp.float32),
        scratch_types=[
            pltpu.VMEM((per_w,), jnp.int32),
            pltpu.VMEM((per_w, emb), jnp.float32),
            pltpu.SemaphoreType.DMA,
        ],
        compiler_params=pltpu.CompilerParams(use_tc_tiling_on_sc=False),
    )
    def gather_k(idx_hbm, table_hbm, out_hbm, idx_v, rows_v, sem):
        wid = lax.axis_index("s") * nc + lax.axis_index("c")
        base = wid * per_w
        pltpu.sync_copy(idx_hbm.at[pl.ds(base, per_w)], idx_v)
        pltpu.async_copy(table_hbm.at[idx_v], rows_v, sem).wait()
        pltpu.sync_copy(rows_v, out_hbm.at[pl.ds(base, per_w)])

    return gather_k


def _lstm_body(emb_ref, wih0_ref, whh0_ref, b0_ref, wih1_ref, whh1_ref,
               b1_ref, hid_ref, cell_ref, h0, c0, h1, c1):
    t = pl.program_id(0)
    nt = pl.num_programs(0)
    H = h0.shape[1]

    @pl.when(t == 0)
    def _init():
        z = jnp.zeros_like(h0)
        h0[...] = z
        c0[...] = z
        h1[...] = z
        c1[...] = z

    def cell_step(x, wih_ref, whh_ref, b_ref, h, c):
        gates = (
            jnp.dot(x, wih_ref[...], preferred_element_type=jnp.float32)
            + jnp.dot(h, whh_ref[...], preferred_element_type=jnp.float32)
            + b_ref[...]
        )
        i = jax.nn.sigmoid(gates[:, 0 * H:1 * H])
        f = jax.nn.sigmoid(gates[:, 1 * H:2 * H])
        g = jnp.tanh(gates[:, 2 * H:3 * H])
        o = jax.nn.sigmoid(gates[:, 3 * H:4 * H])
        c_new = f * c + i * g
        h_new = o * jnp.tanh(c_new)
        return h_new, c_new

    x_t = emb_ref[0]
    h0n, c0n = cell_step(x_t, wih0_ref, whh0_ref, b0_ref, h0[...], c0[...])
    h0[...] = h0n
    c0[...] = c0n
    h1n, c1n = cell_step(h0n, wih1_ref, whh1_ref, b1_ref, h1[...], c1[...])
    h1[...] = h1n
    c1[...] = c1n

    @pl.when(t == nt - 1)
    def _out():
        hid_ref[0] = h0n
        hid_ref[1] = h1n
        cell_ref[0] = c0n
        cell_ref[1] = c1n


def _run_lstm(emb, wih0t, whh0t, b0, wih1t, whh1t, b1):
    T, B, E = emb.shape
    H = whh0t.shape[0]
    full = lambda a: pl.BlockSpec(a.shape, lambda t: (0,) * a.ndim)
    hidden, cell = pl.pallas_call(
        _lstm_body,
        grid=(T,),
        in_specs=[
            pl.BlockSpec((1, B, E), lambda t: (t, 0, 0)),
            full(wih0t), full(whh0t), full(b0),
            full(wih1t), full(whh1t), full(b1),
        ],
        out_specs=[
            pl.BlockSpec((2, B, H), lambda t: (0, 0, 0)),
            pl.BlockSpec((2, B, H), lambda t: (0, 0, 0)),
        ],
        out_shape=[
            jax.ShapeDtypeStruct((2, B, H), jnp.float32),
            jax.ShapeDtypeStruct((2, B, H), jnp.float32),
        ],
        scratch_shapes=[pltpu.VMEM((B, H), jnp.float32) for _ in range(4)],
        compiler_params=pltpu.CompilerParams(
            dimension_semantics=("arbitrary",),
        ),
    )(emb, wih0t, whh0t, b0, wih1t, whh1t, b1)
    return hidden, cell


def kernel(src, emb_table, W_ih0, W_hh0, b_ih0, b_hh0,
           W_ih1, W_hh1, b_ih1, b_hh1):
    B, T = src.shape
    V, E = emb_table.shape
    H = W_hh0.shape[1]

    idx = src.astype(jnp.int32).T.reshape(-1)
    emb_flat = _make_gather(B * T, V, E)(idx, emb_table)
    emb = emb_flat.reshape(T, B, E)

    hidden, cell = _run_lstm(
        emb,
        W_ih0.T, W_hh0.T, (b_ih0 + b_hh0).reshape(1, 4 * H),
        W_ih1.T, W_hh1.T, (b_ih1 + b_hh1).reshape(1, 4 * H),
    )
    return hidden, cell


# speedup vs baseline: 1.0350x; 1.0350x over previous
import functools

import jax
import jax.numpy as jnp
from jax import lax
from jax.experimental import pallas as pl
from jax.experimental.pallas import tpu as pltpu
from jax.experimental.pallas import tpu_sc as plsc


@functools.lru_cache(maxsize=None)
def _make_gather(n_idx: int, vocab: int, emb: int):
    info = plsc.get_sparse_core_info()
    nc, ns = info.num_cores, info.num_subcores
    nw = nc * ns
    assert n_idx % (8 * nw) == 0 and emb == 64
    per_w = n_idx // nw
    wide_per_w = per_w // 2
    mesh = plsc.VectorSubcoreMesh(core_axis_name="c", subcore_axis_name="s")

    half = n_idx // 2

    @functools.partial(
        pl.kernel,
        mesh=mesh,
        out_type=jax.ShapeDtypeStruct((half, 2 * emb), jnp.float32),
        scratch_types=[
            pltpu.VMEM((wide_per_w,), jnp.int32),
            pltpu.VMEM((wide_per_w,), jnp.int32),
            pltpu.VMEM((wide_per_w, emb), jnp.float32),
            pltpu.VMEM((wide_per_w, emb), jnp.float32),
            pltpu.SemaphoreType.DMA,
        ],
        compiler_params=pltpu.CompilerParams(use_tc_tiling_on_sc=False),
    )
    def gather_k(idx_hbm, table_hbm, out_hbm, idx_e, idx_o, rows_e, rows_o,
                 sem):
        wid = lax.axis_index("s") * nc + lax.axis_index("c")
        base = wid * wide_per_w
        pltpu.sync_copy(idx_hbm.at[pl.ds(base, wide_per_w)], idx_e)
        pltpu.sync_copy(idx_hbm.at[pl.ds(half + base, wide_per_w)], idx_o)
        cp_e = pltpu.async_copy(table_hbm.at[idx_e], rows_e, sem)
        cp_o = pltpu.async_copy(table_hbm.at[idx_o], rows_o, sem)
        cp_e.wait()
        cp_o.wait()
        pltpu.sync_copy(rows_e,
                        out_hbm.at[pl.ds(base, wide_per_w), pl.ds(0, emb)])
        pltpu.sync_copy(rows_o,
                        out_hbm.at[pl.ds(base, wide_per_w), pl.ds(emb, emb)])

    return gather_k


def _lstm_body(emb_ref, wih0d_ref, whh0_ref, b0_ref, wih1_ref, whh1_ref,
               b1_ref, hid_ref, cell_ref, h0, c0, h1, c1):
    u = pl.program_id(0)
    nu = pl.num_programs(0)
    H = h0.shape[1]

    @pl.when(u == 0)
    def _init():
        z = jnp.zeros_like(h0)
        h0[...] = z
        c0[...] = z
        h1[...] = z
        c1[...] = z

    xg = jnp.dot(emb_ref[0], wih0d_ref[...],
                 preferred_element_type=jnp.float32)

    def cell_step(gx, whh_ref, b_ref, h, c):
        gates = gx + jnp.dot(h, whh_ref[...],
                             preferred_element_type=jnp.float32) + b_ref[...]
        i = jax.nn.sigmoid(gates[:, 0 * H:1 * H])
        f = jax.nn.sigmoid(gates[:, 1 * H:2 * H])
        g = jnp.tanh(gates[:, 2 * H:3 * H])
        o = jax.nn.sigmoid(gates[:, 3 * H:4 * H])
        c_new = f * c + i * g
        h_new = o * jnp.tanh(c_new)
        return h_new, c_new

    h0c, c0c = h0[...], c0[...]
    h1c, c1c = h1[...], c1[...]
    for half in (0, 1):
        gx0 = xg[:, half * 4 * H:(half + 1) * 4 * H]
        h0c, c0c = cell_step(gx0, whh0_ref, b0_ref, h0c, c0c)
        gx1 = jnp.dot(h0c, wih1_ref[...], preferred_element_type=jnp.float32)
        h1c, c1c = cell_step(gx1, whh1_ref, b1_ref, h1c, c1c)
    h0[...] = h0c
    c0[...] = c0c
    h1[...] = h1c
    c1[...] = c1c

    @pl.when(u == nu - 1)
    def _out():
        hid_ref[0] = h0c
        hid_ref[1] = h1c
        cell_ref[0] = c0c
        cell_ref[1] = c1c


def _run_lstm(emb, wih0d, whh0t, b0, wih1t, whh1t, b1):
    U, B, E2 = emb.shape
    H = whh0t.shape[0]
    full = lambda a: pl.BlockSpec(a.shape, lambda u: (0,) * a.ndim)
    hidden, cell = pl.pallas_call(
        _lstm_body,
        grid=(U,),
        in_specs=[
            pl.BlockSpec((1, B, E2), lambda u: (u, 0, 0)),
            full(wih0d), full(whh0t), full(b0),
            full(wih1t), full(whh1t), full(b1),
        ],
        out_specs=[
            pl.BlockSpec((2, B, H), lambda u: (0, 0, 0)),
            pl.BlockSpec((2, B, H), lambda u: (0, 0, 0)),
        ],
        out_shape=[
            jax.ShapeDtypeStruct((2, B, H), jnp.float32),
            jax.ShapeDtypeStruct((2, B, H), jnp.float32),
        ],
        scratch_shapes=[pltpu.VMEM((B, H), jnp.float32) for _ in range(4)],
        compiler_params=pltpu.CompilerParams(
            dimension_semantics=("arbitrary",),
        ),
    )(emb, wih0d, whh0t, b0, wih1t, whh1t, b1)
    return hidden, cell


def kernel(src, emb_table, W_ih0, W_hh0, b_ih0, b_hh0,
           W_ih1, W_hh1, b_ih1, b_hh1):
    B, T = src.shape
    V, E = emb_table.shape
    H = W_hh0.shape[1]
    U = T // 2

    idx = jnp.transpose(src.astype(jnp.int32).reshape(B, U, 2),
                        (2, 1, 0)).reshape(-1)
    emb_wide = _make_gather(B * T, V, E)(idx, emb_table)
    emb = emb_wide.reshape(U, B, 2 * E)

    wih0t = W_ih0.T
    z = jnp.zeros((E, 4 * H), jnp.float32)
    wih0d = jnp.block([[wih0t, z], [z, wih0t]])

    hidden, cell = _run_lstm(
        emb,
        wih0d, W_hh0.T, (b_ih0 + b_hh0).reshape(1, 4 * H),
        W_ih1.T, W_hh1.T, (b_ih1 + b_hh1).reshape(1, 4 * H),
    )
    return hidden, cell


# speedup vs baseline: 1.0363x; 1.0013x over previous
import functools

import jax
import jax.numpy as jnp
from jax import lax
from jax.experimental import pallas as pl
from jax.experimental.pallas import tpu as pltpu
from jax.experimental.pallas import tpu_sc as plsc


@functools.lru_cache(maxsize=None)
def _make_gather(batch: int, seq: int, vocab: int, emb: int):
    info = plsc.get_sparse_core_info()
    nc, ns, nl = info.num_cores, info.num_subcores, info.num_lanes
    nw = nc * ns
    u_steps = seq // 2
    b_per_w = batch // nw
    per_w = b_per_w * u_steps
    assert batch % nw == 0 and seq % 2 == 0 and per_w % nl == 0
    mesh = plsc.VectorSubcoreMesh(core_axis_name="c", subcore_axis_name="s")

    @functools.partial(
        pl.kernel,
        mesh=mesh,
        out_type=jax.ShapeDtypeStruct((batch * u_steps, 2 * emb),
                                      jnp.float32),
        scratch_types=[
            pltpu.VMEM((b_per_w, seq), jnp.int32),
            pltpu.VMEM((per_w,), jnp.int32),
            pltpu.VMEM((per_w,), jnp.int32),
            pltpu.VMEM((per_w, emb), jnp.float32),
            pltpu.VMEM((per_w, emb), jnp.float32),
            pltpu.SemaphoreType.DMA,
        ],
        compiler_params=pltpu.CompilerParams(use_tc_tiling_on_sc=False,
                                             needs_layout_passes=False),
    )
    def gather_k(src_hbm, table_hbm, out_hbm, idx_v, idx_e, idx_o,
                 rows_e, rows_o, sem):
        wid = lax.axis_index("s") * nc + lax.axis_index("c")
        pltpu.sync_copy(src_hbm.at[pl.ds(wid * b_per_w, b_per_w)], idx_v)

        def perm(chunk, bu):
            b, u = bu
            idx_e[pl.ds(chunk * nl, nl)] = plsc.load_gather(
                idx_v, [b, 2 * u])
            idx_o[pl.ds(chunk * nl, nl)] = plsc.load_gather(
                idx_v, [b, 2 * u + 1])
            un = u + nl
            wrap = (un >= u_steps).astype(jnp.int32)
            return b + wrap, un - wrap * u_steps

        b0 = jnp.zeros((nl,), jnp.int32)
        u0 = lax.iota(jnp.int32, nl)
        lax.fori_loop(0, per_w // nl, perm, (b0, u0))

        cp_e = pltpu.async_copy(table_hbm.at[idx_e], rows_e, sem)
        cp_o = pltpu.async_copy(table_hbm.at[idx_o], rows_o, sem)
        cp_e.wait()
        cp_o.wait()
        base = wid * per_w
        pltpu.sync_copy(rows_e,
                        out_hbm.at[pl.ds(base, per_w), pl.ds(0, emb)])
        pltpu.sync_copy(rows_o,
                        out_hbm.at[pl.ds(base, per_w), pl.ds(emb, emb)])

    return gather_k


def _lstm_body(emb_ref, wih0d_ref, whh0_ref, b0_ref, wih1_ref, whh1_ref,
               b1_ref, hid_ref, cell_ref, h0, c0, h1, c1):
    u = pl.program_id(0)
    nu = pl.num_programs(0)
    H = h0.shape[1]

    @pl.when(u == 0)
    def _init():
        z = jnp.zeros_like(h0)
        h0[...] = z
        c0[...] = z
        h1[...] = z
        c1[...] = z

    xg = jnp.dot(emb_ref[:, 0, 0, :], wih0d_ref[...],
                 preferred_element_type=jnp.float32)

    def cell_step(gx, whh_ref, b_ref, h, c):
        gates = gx + jnp.dot(h, whh_ref[...],
                             preferred_element_type=jnp.float32) + b_ref[...]
        i = jax.nn.sigmoid(gates[:, 0 * H:1 * H])
        f = jax.nn.sigmoid(gates[:, 1 * H:2 * H])
        g = jnp.tanh(gates[:, 2 * H:3 * H])
        o = jax.nn.sigmoid(gates[:, 3 * H:4 * H])
        c_new = f * c + i * g
        h_new = o * jnp.tanh(c_new)
        return h_new, c_new

    h0c, c0c = h0[...], c0[...]
    h1c, c1c = h1[...], c1[...]
    for half in (0, 1):
        gx0 = xg[:, half * 4 * H:(half + 1) * 4 * H]
        h0c, c0c = cell_step(gx0, whh0_ref, b0_ref, h0c, c0c)
        gx1 = jnp.dot(h0c, wih1_ref[...], preferred_element_type=jnp.float32)
        h1c, c1c = cell_step(gx1, whh1_ref, b1_ref, h1c, c1c)
    h0[...] = h0c
    c0[...] = c0c
    h1[...] = h1c
    c1[...] = c1c

    @pl.when(u == nu - 1)
    def _out():
        hid_ref[0] = h0c
        hid_ref[1] = h1c
        cell_ref[0] = c0c
        cell_ref[1] = c1c


def _run_lstm(emb, wih0d, whh0t, b0, wih1t, whh1t, b1):
    B, U, _, E2 = emb.shape
    H = whh0t.shape[0]
    full = lambda a: pl.BlockSpec(a.shape, lambda u: (0,) * a.ndim)
    hidden, cell = pl.pallas_call(
        _lstm_body,
        grid=(U,),
        in_specs=[
            pl.BlockSpec((B, 1, 1, E2), lambda u: (0, u, 0, 0)),
            full(wih0d), full(whh0t), full(b0),
            full(wih1t), full(whh1t), full(b1),
        ],
        out_specs=[
            pl.BlockSpec((2, B, H), lambda u: (0, 0, 0)),
            pl.BlockSpec((2, B, H), lambda u: (0, 0, 0)),
        ],
        out_shape=[
            jax.ShapeDtypeStruct((2, B, H), jnp.float32),
            jax.ShapeDtypeStruct((2, B, H), jnp.float32),
        ],
        scratch_shapes=[pltpu.VMEM((B, H), jnp.float32) for _ in range(4)],
        compiler_params=pltpu.CompilerParams(
            dimension_semantics=("arbitrary",),
        ),
    )(emb, wih0d, whh0t, b0, wih1t, whh1t, b1)
    return hidden, cell


def kernel(src, emb_table, W_ih0, W_hh0, b_ih0, b_hh0,
           W_ih1, W_hh1, b_ih1, b_hh1):
    B, T = src.shape
    V, E = emb_table.shape
    H = W_hh0.shape[1]
    U = T // 2

    emb_wide = _make_gather(B, T, V, E)(src.astype(jnp.int32), emb_table)
    emb = emb_wide.reshape(B, U, 1, 2 * E)

    wih0t = W_ih0.T
    z = jnp.zeros((E, 4 * H), jnp.float32)
    wih0d = jnp.block([[wih0t, z], [z, wih0t]])

    hidden, cell = _run_lstm(
        emb,
        wih0d, W_hh0.T, (b_ih0 + b_hh0).reshape(1, 4 * H),
        W_ih1.T, W_hh1.T, (b_ih1 + b_hh1).reshape(1, 4 * H),
    )
    return hidden, cell


# speedup vs baseline: 1.2099x; 1.1675x over previous
import functools

import jax
import jax.numpy as jnp
from jax import lax
from jax.experimental import pallas as pl
from jax.experimental.pallas import tpu as pltpu
from jax.experimental.pallas import tpu_sc as plsc


_REPACK_LANES = 2048


def _repack_body(tt_ref, out_ref):
    x = tt_ref[...]
    eye = jax.lax.broadcasted_iota(jnp.int32, (64, 64), 0) == \
        jax.lax.broadcasted_iota(jnp.int32, (64, 64), 1)
    xt = jax.lax.dot_general(
        x, eye.astype(jnp.float32), (((0,), (0,)), ((), ())),
        preferred_element_type=jnp.float32)
    out_ref[:, 0:64] = xt
    out_ref[:, 64:128] = jnp.zeros_like(xt)


def _repack_table(table_t):
    e, v = table_t.shape
    lb = _REPACK_LANES
    grid = (v + lb - 1) // lb
    return pl.pallas_call(
        _repack_body,
        grid=(grid,),
        in_specs=[pl.BlockSpec((e, lb), lambda i: (0, i))],
        out_specs=pl.BlockSpec((lb, 2 * e), lambda i: (i, 0)),
        out_shape=jax.ShapeDtypeStruct((v, 2 * e), jnp.float32),
        compiler_params=pltpu.CompilerParams(
            dimension_semantics=("arbitrary",),
        ),
    )(table_t)


@functools.lru_cache(maxsize=None)
def _make_gather(batch: int, seq: int, vocab: int, emb: int):
    info = plsc.get_sparse_core_info()
    nc, ns, nl = info.num_cores, info.num_subcores, info.num_lanes
    nw = nc * ns
    u_steps = seq // 2
    b_per_w = batch // nw
    per_w = b_per_w * u_steps
    assert batch % nw == 0 and seq % 2 == 0 and per_w % nl == 0
    mesh = plsc.VectorSubcoreMesh(core_axis_name="c", subcore_axis_name="s")

    @functools.partial(
        pl.kernel,
        mesh=mesh,
        out_type=jax.ShapeDtypeStruct((batch * u_steps, 2 * emb),
                                      jnp.float32),
        scratch_types=[
            pltpu.VMEM((b_per_w, seq), jnp.int32),
            pltpu.VMEM((per_w,), jnp.int32),
            pltpu.VMEM((per_w,), jnp.int32),
            pltpu.VMEM((per_w, 2 * emb), jnp.float32),
            pltpu.SemaphoreType.DMA,
        ],
        compiler_params=pltpu.CompilerParams(use_tc_tiling_on_sc=False,
                                             needs_layout_passes=False),
    )
    def gather_k(src_hbm, table_hbm, out_hbm, idx_v, idx_e, idx_o,
                 rows_v, sem):
        wid = lax.axis_index("s") * nc + lax.axis_index("c")
        pltpu.sync_copy(src_hbm.at[pl.ds(wid * b_per_w, b_per_w)], idx_v)

        def perm(chunk, bu):
            b, u = bu
            idx_e[pl.ds(chunk * nl, nl)] = plsc.load_gather(
                idx_v, [b, 2 * u])
            idx_o[pl.ds(chunk * nl, nl)] = plsc.load_gather(
                idx_v, [b, 2 * u + 1])
            un = u + nl
            wrap = (un >= u_steps).astype(jnp.int32)
            return b + wrap, un - wrap * u_steps

        b0 = jnp.zeros((nl,), jnp.int32)
        u0 = lax.iota(jnp.int32, nl)
        lax.fori_loop(0, per_w // nl, perm, (b0, u0))

        base = wid * per_w
        pltpu.async_copy(table_hbm.at[idx_e], rows_v, sem).wait()
        pltpu.sync_copy(rows_v.at[:, pl.ds(0, emb)],
                        out_hbm.at[pl.ds(base, per_w), pl.ds(0, emb)])
        pltpu.async_copy(table_hbm.at[idx_o], rows_v, sem).wait()
        pltpu.sync_copy(rows_v.at[:, pl.ds(0, emb)],
                        out_hbm.at[pl.ds(base, per_w), pl.ds(emb, emb)])

    return gather_k


def _lstm_body(emb_ref, wih0d_ref, whh0_ref, b0_ref, wih1_ref, whh1_ref,
               b1_ref, hid_ref, cell_ref, h0, c0, h1, c1):
    u = pl.program_id(0)
    nu = pl.num_programs(0)
    H = h0.shape[1]

    @pl.when(u == 0)
    def _init():
        z = jnp.zeros_like(h0)
        h0[...] = z
        c0[...] = z
        h1[...] = z
        c1[...] = z

    xg = jnp.dot(emb_ref[:, 0, 0, :], wih0d_ref[...],
                 preferred_element_type=jnp.float32)

    def cell_step(gx, whh_ref, b_ref, h, c):
        gates = gx + jnp.dot(h, whh_ref[...],
                             preferred_element_type=jnp.float32) + b_ref[...]
        i = jax.nn.sigmoid(gates[:, 0 * H:1 * H])
        f = jax.nn.sigmoid(gates[:, 1 * H:2 * H])
        g = jnp.tanh(gates[:, 2 * H:3 * H])
        o = jax.nn.sigmoid(gates[:, 3 * H:4 * H])
        c_new = f * c + i * g
        h_new = o * jnp.tanh(c_new)
        return h_new, c_new

    h0c, c0c = h0[...], c0[...]
    h1c, c1c = h1[...], c1[...]
    for half in (0, 1):
        gx0 = xg[:, half * 4 * H:(half + 1) * 4 * H]
        h0c, c0c = cell_step(gx0, whh0_ref, b0_ref, h0c, c0c)
        gx1 = jnp.dot(h0c, wih1_ref[...], preferred_element_type=jnp.float32)
        h1c, c1c = cell_step(gx1, whh1_ref, b1_ref, h1c, c1c)
    h0[...] = h0c
    c0[...] = c0c
    h1[...] = h1c
    c1[...] = c1c

    @pl.when(u == nu - 1)
    def _out():
        hid_ref[0] = h0c
        hid_ref[1] = h1c
        cell_ref[0] = c0c
        cell_ref[1] = c1c


def _run_lstm(emb, wih0d, whh0t, b0, wih1t, whh1t, b1):
    B, U, _, E2 = emb.shape
    H = whh0t.shape[0]
    full = lambda a: pl.BlockSpec(a.shape, lambda u: (0,) * a.ndim)
    hidden, cell = pl.pallas_call(
        _lstm_body,
        grid=(U,),
        in_specs=[
            pl.BlockSpec((B, 1, 1, E2), lambda u: (0, u, 0, 0)),
            full(wih0d), full(whh0t), full(b0),
            full(wih1t), full(whh1t), full(b1),
        ],
        out_specs=[
            pl.BlockSpec((2, B, H), lambda u: (0, 0, 0)),
            pl.BlockSpec((2, B, H), lambda u: (0, 0, 0)),
        ],
        out_shape=[
            jax.ShapeDtypeStruct((2, B, H), jnp.float32),
            jax.ShapeDtypeStruct((2, B, H), jnp.float32),
        ],
        scratch_shapes=[pltpu.VMEM((B, H), jnp.float32) for _ in range(4)],
        compiler_params=pltpu.CompilerParams(
            dimension_semantics=("arbitrary",),
        ),
    )(emb, wih0d, whh0t, b0, wih1t, whh1t, b1)
    return hidden, cell


def kernel(src, emb_table, W_ih0, W_hh0, b_ih0, b_hh0,
           W_ih1, W_hh1, b_ih1, b_hh1):
    B, T = src.shape
    V, E = emb_table.shape
    H = W_hh0.shape[1]
    U = T // 2

    table_lin = _repack_table(emb_table.T)
    emb_wide = _make_gather(B, T, V, E)(src.astype(jnp.int32), table_lin)
    emb = emb_wide.reshape(B, U, 1, 2 * E)

    wih0t = W_ih0.T
    z = jnp.zeros((E, 4 * H), jnp.float32)
    wih0d = jnp.block([[wih0t, z], [z, wih0t]])

    hidden, cell = _run_lstm(
        emb,
        wih0d, W_hh0.T, (b_ih0 + b_hh0).reshape(1, 4 * H),
        W_ih1.T, W_hh1.T, (b_ih1 + b_hh1).reshape(1, 4 * H),
    )
    return hidden, cell


# speedup vs baseline: 1.6895x; 1.3964x over previous
import functools

import jax
import jax.numpy as jnp
from jax import lax
from jax.experimental import pallas as pl
from jax.experimental.pallas import tpu as pltpu
from jax.experimental.pallas import tpu_sc as plsc


_REPACK_LANES = 2048


def _repack_body(lo_ref, hi_ref, out_ref):
    xt_lo = jnp.swapaxes(lo_ref[...], 0, 1)
    xt_hi = jnp.swapaxes(hi_ref[...], 0, 1)
    out_ref[:, 0:64] = xt_lo
    out_ref[:, 64:128] = xt_hi


def _repack_table(table_t):
    e, v = table_t.shape
    lb = _REPACK_LANES
    nblk = -(-v // (2 * lb))
    maxb = (v - 1) // lb
    return pl.pallas_call(
        _repack_body,
        grid=(nblk,),
        in_specs=[
            pl.BlockSpec((e, lb), lambda i: (0, 2 * i)),
            pl.BlockSpec((e, lb),
                         lambda i: (0, jnp.minimum(2 * i + 1, maxb))),
        ],
        out_specs=pl.BlockSpec((lb, 2 * e), lambda i: (i, 0)),
        out_shape=jax.ShapeDtypeStruct((nblk * lb, 2 * e), jnp.float32),
        compiler_params=pltpu.CompilerParams(
            dimension_semantics=("arbitrary",),
        ),
    )(table_t, table_t)


@functools.lru_cache(maxsize=None)
def _make_gather(batch: int, seq: int, view_rows: int, emb: int):
    info = plsc.get_sparse_core_info()
    nc, ns, nl = info.num_cores, info.num_subcores, info.num_lanes
    nw = nc * ns
    u_steps = seq // 2
    b_per_w = batch // nw
    per_w = b_per_w * u_steps
    assert batch % nw == 0 and seq % 2 == 0 and per_w % nl == 0
    mesh = plsc.VectorSubcoreMesh(core_axis_name="c", subcore_axis_name="s")
    lb = _REPACK_LANES

    @functools.partial(
        pl.kernel,
        mesh=mesh,
        out_type=jax.ShapeDtypeStruct((batch * u_steps, 2 * emb),
                                      jnp.float32),
        scratch_types=[
            pltpu.VMEM((b_per_w, seq), jnp.int32),
            pltpu.VMEM((per_w,), jnp.int32),
            pltpu.VMEM((per_w,), jnp.int32),
            pltpu.VMEM((per_w, emb), jnp.float32),
            pltpu.VMEM((per_w, emb), jnp.float32),
            pltpu.SemaphoreType.DMA,
        ],
        compiler_params=pltpu.CompilerParams(use_tc_tiling_on_sc=False,
                                             needs_layout_passes=False),
    )
    def gather_k(src_hbm, table_hbm, out_hbm, idx_v, idx_e, idx_o,
                 rows_e, rows_o, sem):
        wid = lax.axis_index("s") * nc + lax.axis_index("c")
        pltpu.sync_copy(src_hbm.at[pl.ds(wid * b_per_w, b_per_w)], idx_v)

        sh = lb.bit_length() - 1

        def view_row(i):
            blk_pair = lax.shift_left(
                lax.shift_right_logical(i, sh + 1), sh + 1)
            half = lax.shift_right_logical(i, sh) & 1
            return lax.shift_left(i & (lb - 1), 1) + blk_pair + half

        def perm(chunk, bu):
            b, u = bu
            idx_e[pl.ds(chunk * nl, nl)] = view_row(
                plsc.load_gather(idx_v, [b, 2 * u]))
            idx_o[pl.ds(chunk * nl, nl)] = view_row(
                plsc.load_gather(idx_v, [b, 2 * u + 1]))
            un = u + nl
            wrap = (un >= u_steps).astype(jnp.int32)
            return b + wrap, un - wrap * u_steps

        b0 = jnp.zeros((nl,), jnp.int32)
        u0 = lax.iota(jnp.int32, nl)
        lax.fori_loop(0, per_w // nl, perm, (b0, u0))

        cp_e = pltpu.async_copy(table_hbm.at[idx_e], rows_e, sem)
        cp_o = pltpu.async_copy(table_hbm.at[idx_o], rows_o, sem)
        cp_e.wait()
        cp_o.wait()
        base = wid * per_w
        pltpu.sync_copy(rows_e,
                        out_hbm.at[pl.ds(base, per_w), pl.ds(0, emb)])
        pltpu.sync_copy(rows_o,
                        out_hbm.at[pl.ds(base, per_w), pl.ds(emb, emb)])

    return gather_k


def _lstm_body(emb_ref, wih0d_ref, whh0_ref, b0_ref, wih1_ref, whh1_ref,
               b1_ref, hid_ref, cell_ref, h0, c0, h1, c1):
    u = pl.program_id(0)
    nu = pl.num_programs(0)
    H = h0.shape[1]

    @pl.when(u == 0)
    def _init():
        z = jnp.zeros_like(h0)
        h0[...] = z
        c0[...] = z
        h1[...] = z
        c1[...] = z

    xg = jnp.dot(emb_ref[:, 0, 0, :], wih0d_ref[...],
                 preferred_element_type=jnp.float32)

    def cell_step(gx, whh_ref, b_ref, h, c):
        gates = gx + jnp.dot(h, whh_ref[...],
                             preferred_element_type=jnp.float32) + b_ref[...]
        i = jax.nn.sigmoid(gates[:, 0 * H:1 * H])
        f = jax.nn.sigmoid(gates[:, 1 * H:2 * H])
        g = jnp.tanh(gates[:, 2 * H:3 * H])
        o = jax.nn.sigmoid(gates[:, 3 * H:4 * H])
        c_new = f * c + i * g
        h_new = o * jnp.tanh(c_new)
        return h_new, c_new

    h0c, c0c = h0[...], c0[...]
    h1c, c1c = h1[...], c1[...]
    for half in (0, 1):
        gx0 = xg[:, half * 4 * H:(half + 1) * 4 * H]
        h0c, c0c = cell_step(gx0, whh0_ref, b0_ref, h0c, c0c)
        gx1 = jnp.dot(h0c, wih1_ref[...], preferred_element_type=jnp.float32)
        h1c, c1c = cell_step(gx1, whh1_ref, b1_ref, h1c, c1c)
    h0[...] = h0c
    c0[...] = c0c
    h1[...] = h1c
    c1[...] = c1c

    @pl.when(u == nu - 1)
    def _out():
        hid_ref[0] = h0c
        hid_ref[1] = h1c
        cell_ref[0] = c0c
        cell_ref[1] = c1c


def _run_lstm(emb, wih0d, whh0t, b0, wih1t, whh1t, b1):
    B, U, _, E2 = emb.shape
    H = whh0t.shape[0]
    full = lambda a: pl.BlockSpec(a.shape, lambda u: (0,) * a.ndim)
    hidden, cell = pl.pallas_call(
        _lstm_body,
        grid=(U,),
        in_specs=[
            pl.BlockSpec((B, 1, 1, E2), lambda u: (0, u, 0, 0)),
            full(wih0d), full(whh0t), full(b0),
            full(wih1t), full(whh1t), full(b1),
        ],
        out_specs=[
            pl.BlockSpec((2, B, H), lambda u: (0, 0, 0)),
            pl.BlockSpec((2, B, H), lambda u: (0, 0, 0)),
        ],
        out_shape=[
            jax.ShapeDtypeStruct((2, B, H), jnp.float32),
            jax.ShapeDtypeStruct((2, B, H), jnp.float32),
        ],
        scratch_shapes=[pltpu.VMEM((B, H), jnp.float32) for _ in range(4)],
        compiler_params=pltpu.CompilerParams(
            dimension_semantics=("arbitrary",),
        ),
    )(emb, wih0d, whh0t, b0, wih1t, whh1t, b1)
    return hidden, cell


def kernel(src, emb_table, W_ih0, W_hh0, b_ih0, b_hh0,
           W_ih1, W_hh1, b_ih1, b_hh1):
    B, T = src.shape
    V, E = emb_table.shape
    H = W_hh0.shape[1]
    U = T // 2

    table_pairs = _repack_table(emb_table.T)
    table_view = table_pairs.reshape(2 * table_pairs.shape[0], E)
    emb_wide = _make_gather(B, T, table_view.shape[0], E)(
        src.astype(jnp.int32), table_view)
    emb = emb_wide.reshape(B, U, 1, 2 * E)

    wih0t = W_ih0.T
    z = jnp.zeros((E, 4 * H), jnp.float32)
    wih0d = jnp.block([[wih0t, z], [z, wih0t]])

    hidden, cell = _run_lstm(
        emb,
        wih0d, W_hh0.T, (b_ih0 + b_hh0).reshape(1, 4 * H),
        W_ih1.T, W_hh1.T, (b_ih1 + b_hh1).reshape(1, 4 * H),
    )
    return hidden, cell


# speedup vs baseline: 1.9999x; 1.1838x over previous
import functools

import jax
import jax.numpy as jnp
from jax import lax
from jax.experimental import pallas as pl
from jax.experimental.pallas import tpu as pltpu
from jax.experimental.pallas import tpu_sc as plsc


_REPACK_LANES = 4096


def _repack_body(lo_ref, hi_ref, out_ref):
    xt_lo = jnp.swapaxes(lo_ref[...], 0, 1)
    xt_hi = jnp.swapaxes(hi_ref[...], 0, 1)
    out_ref[:, 0:64] = xt_lo
    out_ref[:, 64:128] = xt_hi


def _repack_table(table_t):
    e, v = table_t.shape
    lb = _REPACK_LANES
    nblk = -(-v // (2 * lb))
    maxb = (v - 1) // lb
    return pl.pallas_call(
        _repack_body,
        grid=(nblk,),
        in_specs=[
            pl.BlockSpec((e, lb), lambda i: (0, 2 * i)),
            pl.BlockSpec((e, lb),
                         lambda i: (0, jnp.minimum(2 * i + 1, maxb))),
        ],
        out_specs=pl.BlockSpec((lb, 2 * e), lambda i: (i, 0)),
        out_shape=jax.ShapeDtypeStruct((nblk * lb, 2 * e), jnp.float32),
        compiler_params=pltpu.CompilerParams(
            dimension_semantics=("arbitrary",),
        ),
    )(table_t, table_t)


@functools.lru_cache(maxsize=None)
def _make_gather(batch: int, seq: int, view_rows: int, emb: int):
    info = plsc.get_sparse_core_info()
    nc, ns, nl = info.num_cores, info.num_subcores, info.num_lanes
    nw = nc * ns
    u_steps = seq // 2
    b_per_w = batch // nw
    per_w = b_per_w * u_steps
    assert batch % nw == 0 and seq % 2 == 0 and per_w % nl == 0
    mesh = plsc.VectorSubcoreMesh(core_axis_name="c", subcore_axis_name="s")
    lb = _REPACK_LANES

    @functools.partial(
        pl.kernel,
        mesh=mesh,
        out_type=jax.ShapeDtypeStruct((batch * u_steps, 2 * emb),
                                      jnp.float32),
        scratch_types=[
            pltpu.VMEM((b_per_w, seq), jnp.int32),
            pltpu.VMEM((per_w,), jnp.int32),
            pltpu.VMEM((per_w,), jnp.int32),
            pltpu.VMEM((per_w, emb), jnp.float32),
            pltpu.VMEM((per_w, emb), jnp.float32),
            pltpu.SemaphoreType.DMA,
        ],
        compiler_params=pltpu.CompilerParams(use_tc_tiling_on_sc=False,
                                             needs_layout_passes=False),
    )
    def gather_k(src_hbm, table_hbm, out_hbm, idx_v, idx_e, idx_o,
                 rows_e, rows_o, sem):
        wid = lax.axis_index("s") * nc + lax.axis_index("c")
        pltpu.sync_copy(src_hbm.at[pl.ds(wid * b_per_w, b_per_w)], idx_v)

        sh = lb.bit_length() - 1

        def view_row(i):
            blk_pair = lax.shift_left(
                lax.shift_right_logical(i, sh + 1), sh + 1)
            half = lax.shift_right_logical(i, sh) & 1
            return lax.shift_left(i & (lb - 1), 1) + blk_pair + half

        def perm(chunk, bu):
            b, u = bu
            idx_e[pl.ds(chunk * nl, nl)] = view_row(
                plsc.load_gather(idx_v, [b, 2 * u]))
            idx_o[pl.ds(chunk * nl, nl)] = view_row(
                plsc.load_gather(idx_v, [b, 2 * u + 1]))
            un = u + nl
            wrap = (un >= u_steps).astype(jnp.int32)
            return b + wrap, un - wrap * u_steps

        b0 = jnp.zeros((nl,), jnp.int32)
        u0 = lax.iota(jnp.int32, nl)
        lax.fori_loop(0, per_w // nl, perm, (b0, u0))

        cp_e = pltpu.async_copy(table_hbm.at[idx_e], rows_e, sem)
        cp_o = pltpu.async_copy(table_hbm.at[idx_o], rows_o, sem)
        cp_e.wait()
        cp_o.wait()
        base = wid * per_w
        pltpu.sync_copy(rows_e,
                        out_hbm.at[pl.ds(base, per_w), pl.ds(0, emb)])
        pltpu.sync_copy(rows_o,
                        out_hbm.at[pl.ds(base, per_w), pl.ds(emb, emb)])

    return gather_k


def _lstm_body(emb_ref, wih0d_ref, whh0_ref, b0_ref, wih1_ref, whh1_ref,
               b1_ref, hid_ref, cell_ref, h0, c0, h1, c1):
    u = pl.program_id(0)
    nu = pl.num_programs(0)
    H = h0.shape[1]

    @pl.when(u == 0)
    def _init():
        z = jnp.zeros_like(h0)
        h0[...] = z
        c0[...] = z
        h1[...] = z
        c1[...] = z

    xg = jnp.dot(emb_ref[:, 0, 0, :], wih0d_ref[...],
                 preferred_element_type=jnp.float32)

    def cell_step(gx, whh_ref, b_ref, h, c):
        gates = gx + jnp.dot(h, whh_ref[...],
                             preferred_element_type=jnp.float32) + b_ref[...]
        i = jax.nn.sigmoid(gates[:, 0 * H:1 * H])
        f = jax.nn.sigmoid(gates[:, 1 * H:2 * H])
        g = jnp.tanh(gates[:, 2 * H:3 * H])
        o = jax.nn.sigmoid(gates[:, 3 * H:4 * H])
        c_new = f * c + i * g
        h_new = o * jnp.tanh(c_new)
        return h_new, c_new

    h0c, c0c = h0[...], c0[...]
    h1c, c1c = h1[...], c1[...]
    for half in (0, 1):
        gx0 = xg[:, half * 4 * H:(half + 1) * 4 * H]
        h0c, c0c = cell_step(gx0, whh0_ref, b0_ref, h0c, c0c)
        gx1 = jnp.dot(h0c, wih1_ref[...], preferred_element_type=jnp.float32)
        h1c, c1c = cell_step(gx1, whh1_ref, b1_ref, h1c, c1c)
    h0[...] = h0c
    c0[...] = c0c
    h1[...] = h1c
    c1[...] = c1c

    @pl.when(u == nu - 1)
    def _out():
        hid_ref[0] = h0c
        hid_ref[1] = h1c
        cell_ref[0] = c0c
        cell_ref[1] = c1c


def _run_lstm(emb, wih0d, whh0t, b0, wih1t, whh1t, b1):
    B, U, _, E2 = emb.shape
    H = whh0t.shape[0]
    full = lambda a: pl.BlockSpec(a.shape, lambda u: (0,) * a.ndim)
    hidden, cell = pl.pallas_call(
        _lstm_body,
        grid=(U,),
        in_specs=[
            pl.BlockSpec((B, 1, 1, E2), lambda u: (0, u, 0, 0)),
            full(wih0d), full(whh0t), full(b0),
            full(wih1t), full(whh1t), full(b1),
        ],
        out_specs=[
            pl.BlockSpec((2, B, H), lambda u: (0, 0, 0)),
            pl.BlockSpec((2, B, H), lambda u: (0, 0, 0)),
        ],
        out_shape=[
            jax.ShapeDtypeStruct((2, B, H), jnp.float32),
            jax.ShapeDtypeStruct((2, B, H), jnp.float32),
        ],
        scratch_shapes=[pltpu.VMEM((B, H), jnp.float32) for _ in range(4)],
        compiler_params=pltpu.CompilerParams(
            dimension_semantics=("arbitrary",),
        ),
    )(emb, wih0d, whh0t, b0, wih1t, whh1t, b1)
    return hidden, cell


def kernel(src, emb_table, W_ih0, W_hh0, b_ih0, b_hh0,
           W_ih1, W_hh1, b_ih1, b_hh1):
    B, T = src.shape
    V, E = emb_table.shape
    H = W_hh0.shape[1]
    U = T // 2

    table_pairs = _repack_table(emb_table.T)
    table_view = table_pairs.reshape(2 * table_pairs.shape[0], E)
    emb_wide = _make_gather(B, T, table_view.shape[0], E)(
        src.astype(jnp.int32), table_view)
    emb = emb_wide.reshape(B, U, 1, 2 * E)

    wih0t = W_ih0.T
    z = jnp.zeros((E, 4 * H), jnp.float32)
    wih0d = jnp.block([[wih0t, z], [z, wih0t]])

    hidden, cell = _run_lstm(
        emb,
        wih0d, W_hh0.T, (b_ih0 + b_hh0).reshape(1, 4 * H),
        W_ih1.T, W_hh1.T, (b_ih1 + b_hh1).reshape(1, 4 * H),
    )
    return hidden, cell


# speedup vs baseline: 2.1952x; 1.0976x over previous
import functools

import jax
import jax.numpy as jnp
from jax import lax
from jax.experimental import pallas as pl
from jax.experimental.pallas import tpu as pltpu
from jax.experimental.pallas import tpu_sc as plsc


_REPACK_LANES = 8192


def _repack_body(lo_ref, hi_ref, out_ref):
    xt_lo = jnp.swapaxes(lo_ref[...], 0, 1)
    xt_hi = jnp.swapaxes(hi_ref[...], 0, 1)
    out_ref[:, 0:64] = xt_lo
    out_ref[:, 64:128] = xt_hi


def _repack_table(table_t):
    e, v = table_t.shape
    lb = _REPACK_LANES
    nblk = -(-v // (2 * lb))
    maxb = (v - 1) // lb
    return pl.pallas_call(
        _repack_body,
        grid=(nblk,),
        in_specs=[
            pl.BlockSpec((e, lb), lambda i: (0, 2 * i)),
            pl.BlockSpec((e, lb),
                         lambda i: (0, jnp.minimum(2 * i + 1, maxb))),
        ],
        out_specs=pl.BlockSpec((lb, 2 * e), lambda i: (i, 0)),
        out_shape=jax.ShapeDtypeStruct((nblk * lb, 2 * e), jnp.float32),
        compiler_params=pltpu.CompilerParams(
            dimension_semantics=("arbitrary",),
        ),
    )(table_t, table_t)


@functools.lru_cache(maxsize=None)
def _make_gather(batch: int, seq: int, view_rows: int, emb: int):
    info = plsc.get_sparse_core_info()
    nc, ns, nl = info.num_cores, info.num_subcores, info.num_lanes
    nw = nc * ns
    u_steps = seq // 2
    b_per_w = batch // nw
    per_w = b_per_w * u_steps
    assert batch % nw == 0 and seq % 2 == 0 and per_w % nl == 0
    mesh = plsc.VectorSubcoreMesh(core_axis_name="c", subcore_axis_name="s")
    lb = _REPACK_LANES

    @functools.partial(
        pl.kernel,
        mesh=mesh,
        out_type=jax.ShapeDtypeStruct((batch * u_steps, 2 * emb),
                                      jnp.float32),
        scratch_types=[
            pltpu.VMEM((b_per_w, seq), jnp.int32),
            pltpu.VMEM((per_w,), jnp.int32),
            pltpu.VMEM((per_w,), jnp.int32),
            pltpu.VMEM((per_w, emb), jnp.float32),
            pltpu.VMEM((per_w, emb), jnp.float32),
            pltpu.SemaphoreType.DMA,
        ],
        compiler_params=pltpu.CompilerParams(use_tc_tiling_on_sc=False,
                                             needs_layout_passes=False),
    )
    def gather_k(src_hbm, table_hbm, out_hbm, idx_v, idx_e, idx_o,
                 rows_e, rows_o, sem):
        wid = lax.axis_index("s") * nc + lax.axis_index("c")
        pltpu.sync_copy(src_hbm.at[pl.ds(wid * b_per_w, b_per_w)], idx_v)

        sh = lb.bit_length() - 1

        def view_row(i):
            blk_pair = lax.shift_left(
                lax.shift_right_logical(i, sh + 1), sh + 1)
            half = lax.shift_right_logical(i, sh) & 1
            return lax.shift_left(i & (lb - 1), 1) + blk_pair + half

        def perm(chunk, bu):
            b, u = bu
            idx_e[pl.ds(chunk * nl, nl)] = view_row(
                plsc.load_gather(idx_v, [b, 2 * u]))
            idx_o[pl.ds(chunk * nl, nl)] = view_row(
                plsc.load_gather(idx_v, [b, 2 * u + 1]))
            un = u + nl
            wrap = (un >= u_steps).astype(jnp.int32)
            return b + wrap, un - wrap * u_steps

        b0 = jnp.zeros((nl,), jnp.int32)
        u0 = lax.iota(jnp.int32, nl)
        lax.fori_loop(0, per_w // nl, perm, (b0, u0))

        cp_e = pltpu.async_copy(table_hbm.at[idx_e], rows_e, sem)
        cp_o = pltpu.async_copy(table_hbm.at[idx_o], rows_o, sem)
        cp_e.wait()
        cp_o.wait()
        base = wid * per_w
        pltpu.sync_copy(rows_e,
                        out_hbm.at[pl.ds(base, per_w), pl.ds(0, emb)])
        pltpu.sync_copy(rows_o,
                        out_hbm.at[pl.ds(base, per_w), pl.ds(emb, emb)])

    return gather_k


def _lstm_body(emb_ref, wih0d_ref, whh0_ref, b0_ref, wih1_ref, whh1_ref,
               b1_ref, hid_ref, cell_ref, h0, c0, h1, c1):
    u = pl.program_id(0)
    nu = pl.num_programs(0)
    H = h0.shape[1]

    @pl.when(u == 0)
    def _init():
        z = jnp.zeros_like(h0)
        h0[...] = z
        c0[...] = z
        h1[...] = z
        c1[...] = z

    xg = jnp.dot(emb_ref[:, 0, 0, :], wih0d_ref[...],
                 preferred_element_type=jnp.float32)

    def cell_step(gx, whh_ref, b_ref, h, c):
        gates = gx + jnp.dot(h, whh_ref[...],
                             preferred_element_type=jnp.float32) + b_ref[...]
        i = jax.nn.sigmoid(gates[:, 0 * H:1 * H])
        f = jax.nn.sigmoid(gates[:, 1 * H:2 * H])
        g = jnp.tanh(gates[:, 2 * H:3 * H])
        o = jax.nn.sigmoid(gates[:, 3 * H:4 * H])
        c_new = f * c + i * g
        h_new = o * jnp.tanh(c_new)
        return h_new, c_new

    h0c, c0c = h0[...], c0[...]
    h1c, c1c = h1[...], c1[...]
    for half in (0, 1):
        gx0 = xg[:, half * 4 * H:(half + 1) * 4 * H]
        h0c, c0c = cell_step(gx0, whh0_ref, b0_ref, h0c, c0c)
        gx1 = jnp.dot(h0c, wih1_ref[...], preferred_element_type=jnp.float32)
        h1c, c1c = cell_step(gx1, whh1_ref, b1_ref, h1c, c1c)
    h0[...] = h0c
    c0[...] = c0c
    h1[...] = h1c
    c1[...] = c1c

    @pl.when(u == nu - 1)
    def _out():
        hid_ref[0] = h0c
        hid_ref[1] = h1c
        cell_ref[0] = c0c
        cell_ref[1] = c1c


def _run_lstm(emb, wih0d, whh0t, b0, wih1t, whh1t, b1):
    B, U, _, E2 = emb.shape
    H = whh0t.shape[0]
    full = lambda a: pl.BlockSpec(a.shape, lambda u: (0,) * a.ndim)
    hidden, cell = pl.pallas_call(
        _lstm_body,
        grid=(U,),
        in_specs=[
            pl.BlockSpec((B, 1, 1, E2), lambda u: (0, u, 0, 0)),
            full(wih0d), full(whh0t), full(b0),
            full(wih1t), full(whh1t), full(b1),
        ],
        out_specs=[
            pl.BlockSpec((2, B, H), lambda u: (0, 0, 0)),
            pl.BlockSpec((2, B, H), lambda u: (0, 0, 0)),
        ],
        out_shape=[
            jax.ShapeDtypeStruct((2, B, H), jnp.float32),
            jax.ShapeDtypeStruct((2, B, H), jnp.float32),
        ],
        scratch_shapes=[pltpu.VMEM((B, H), jnp.float32) for _ in range(4)],
        compiler_params=pltpu.CompilerParams(
            dimension_semantics=("arbitrary",),
        ),
    )(emb, wih0d, whh0t, b0, wih1t, whh1t, b1)
    return hidden, cell


def kernel(src, emb_table, W_ih0, W_hh0, b_ih0, b_hh0,
           W_ih1, W_hh1, b_ih1, b_hh1):
    B, T = src.shape
    V, E = emb_table.shape
    H = W_hh0.shape[1]
    U = T // 2

    table_pairs = _repack_table(emb_table.T)
    table_view = table_pairs.reshape(2 * table_pairs.shape[0], E)
    emb_wide = _make_gather(B, T, table_view.shape[0], E)(
        src.astype(jnp.int32), table_view)
    emb = emb_wide.reshape(B, U, 1, 2 * E)

    wih0t = W_ih0.T
    z = jnp.zeros((E, 4 * H), jnp.float32)
    wih0d = jnp.block([[wih0t, z], [z, wih0t]])

    hidden, cell = _run_lstm(
        emb,
        wih0d, W_hh0.T, (b_ih0 + b_hh0).reshape(1, 4 * H),
        W_ih1.T, W_hh1.T, (b_ih1 + b_hh1).reshape(1, 4 * H),
    )
    return hidden, cell


# speedup vs baseline: 2.2745x; 1.0361x over previous
import functools

import jax
import jax.numpy as jnp
from jax import lax
from jax.experimental import pallas as pl
from jax.experimental.pallas import tpu as pltpu
from jax.experimental.pallas import tpu_sc as plsc


_REPACK_LANES = 16384


def _repack_body(lo_ref, hi_ref, out_ref):
    xt_lo = jnp.swapaxes(lo_ref[...], 0, 1)
    xt_hi = jnp.swapaxes(hi_ref[...], 0, 1)
    out_ref[:, 0:64] = xt_lo
    out_ref[:, 64:128] = xt_hi


def _repack_table(table_t):
    e, v = table_t.shape
    lb = _REPACK_LANES
    nblk = -(-v // (2 * lb))
    maxb = (v - 1) // lb
    return pl.pallas_call(
        _repack_body,
        grid=(nblk,),
        in_specs=[
            pl.BlockSpec((e, lb), lambda i: (0, 2 * i)),
            pl.BlockSpec((e, lb),
                         lambda i: (0, jnp.minimum(2 * i + 1, maxb))),
        ],
        out_specs=pl.BlockSpec((lb, 2 * e), lambda i: (i, 0)),
        out_shape=jax.ShapeDtypeStruct((nblk * lb, 2 * e), jnp.float32),
        compiler_params=pltpu.CompilerParams(
            dimension_semantics=("arbitrary",),
            vmem_limit_bytes=100 * 1024 * 1024,
        ),
    )(table_t, table_t)


@functools.lru_cache(maxsize=None)
def _make_gather(batch: int, seq: int, view_rows: int, emb: int):
    info = plsc.get_sparse_core_info()
    nc, ns, nl = info.num_cores, info.num_subcores, info.num_lanes
    nw = nc * ns
    u_steps = seq // 2
    b_per_w = batch // nw
    per_w = b_per_w * u_steps
    assert batch % nw == 0 and seq % 2 == 0 and per_w % nl == 0
    mesh = plsc.VectorSubcoreMesh(core_axis_name="c", subcore_axis_name="s")
    lb = _REPACK_LANES

    @functools.partial(
        pl.kernel,
        mesh=mesh,
        out_type=jax.ShapeDtypeStruct((batch * u_steps, 2 * emb),
                                      jnp.float32),
        scratch_types=[
            pltpu.VMEM((b_per_w, seq), jnp.int32),
            pltpu.VMEM((per_w,), jnp.int32),
            pltpu.VMEM((per_w,), jnp.int32),
            pltpu.VMEM((per_w, emb), jnp.float32),
            pltpu.VMEM((per_w, emb), jnp.float32),
            pltpu.SemaphoreType.DMA,
        ],
        compiler_params=pltpu.CompilerParams(use_tc_tiling_on_sc=False,
                                             needs_layout_passes=False),
    )
    def gather_k(src_hbm, table_hbm, out_hbm, idx_v, idx_e, idx_o,
                 rows_e, rows_o, sem):
        wid = lax.axis_index("s") * nc + lax.axis_index("c")
        pltpu.sync_copy(src_hbm.at[pl.ds(wid * b_per_w, b_per_w)], idx_v)

        sh = lb.bit_length() - 1

        def view_row(i):
            blk_pair = lax.shift_left(
                lax.shift_right_logical(i, sh + 1), sh + 1)
            half = lax.shift_right_logical(i, sh) & 1
            return lax.shift_left(i & (lb - 1), 1) + blk_pair + half

        def perm(chunk, bu):
            b, u = bu
            idx_e[pl.ds(chunk * nl, nl)] = view_row(
                plsc.load_gather(idx_v, [b, 2 * u]))
            idx_o[pl.ds(chunk * nl, nl)] = view_row(
                plsc.load_gather(idx_v, [b, 2 * u + 1]))
            un = u + nl
            wrap = (un >= u_steps).astype(jnp.int32)
            return b + wrap, un - wrap * u_steps

        b0 = jnp.zeros((nl,), jnp.int32)
        u0 = lax.iota(jnp.int32, nl)
        lax.fori_loop(0, per_w // nl, perm, (b0, u0))

        cp_e = pltpu.async_copy(table_hbm.at[idx_e], rows_e, sem)
        cp_o = pltpu.async_copy(table_hbm.at[idx_o], rows_o, sem)
        cp_e.wait()
        cp_o.wait()
        base = wid * per_w
        pltpu.sync_copy(rows_e,
                        out_hbm.at[pl.ds(base, per_w), pl.ds(0, emb)])
        pltpu.sync_copy(rows_o,
                        out_hbm.at[pl.ds(base, per_w), pl.ds(emb, emb)])

    return gather_k


def _lstm_body(emb_ref, wih0d_ref, whh0_ref, b0_ref, wih1_ref, whh1_ref,
               b1_ref, hid_ref, cell_ref, h0, c0, h1, c1):
    u = pl.program_id(0)
    nu = pl.num_programs(0)
    H = h0.shape[1]

    @pl.when(u == 0)
    def _init():
        z = jnp.zeros_like(h0)
        h0[...] = z
        c0[...] = z
        h1[...] = z
        c1[...] = z

    xg = jnp.dot(emb_ref[:, 0, 0, :], wih0d_ref[...],
                 preferred_element_type=jnp.float32)

    def cell_step(gx, whh_ref, b_ref, h, c):
        gates = gx + jnp.dot(h, whh_ref[...],
                             preferred_element_type=jnp.float32) + b_ref[...]
        i = jax.nn.sigmoid(gates[:, 0 * H:1 * H])
        f = jax.nn.sigmoid(gates[:, 1 * H:2 * H])
        g = jnp.tanh(gates[:, 2 * H:3 * H])
        o = jax.nn.sigmoid(gates[:, 3 * H:4 * H])
        c_new = f * c + i * g
        h_new = o * jnp.tanh(c_new)
        return h_new, c_new

    h0c, c0c = h0[...], c0[...]
    h1c, c1c = h1[...], c1[...]
    for half in (0, 1):
        gx0 = xg[:, half * 4 * H:(half + 1) * 4 * H]
        h0c, c0c = cell_step(gx0, whh0_ref, b0_ref, h0c, c0c)
        gx1 = jnp.dot(h0c, wih1_ref[...], preferred_element_type=jnp.float32)
        h1c, c1c = cell_step(gx1, whh1_ref, b1_ref, h1c, c1c)
    h0[...] = h0c
    c0[...] = c0c
    h1[...] = h1c
    c1[...] = c1c

    @pl.when(u == nu - 1)
    def _out():
        hid_ref[0] = h0c
        hid_ref[1] = h1c
        cell_ref[0] = c0c
        cell_ref[1] = c1c


def _run_lstm(emb, wih0d, whh0t, b0, wih1t, whh1t, b1):
    B, U, _, E2 = emb.shape
    H = whh0t.shape[0]
    full = lambda a: pl.BlockSpec(a.shape, lambda u: (0,) * a.ndim)
    hidden, cell = pl.pallas_call(
        _lstm_body,
        grid=(U,),
        in_specs=[
            pl.BlockSpec((B, 1, 1, E2), lambda u: (0, u, 0, 0)),
            full(wih0d), full(whh0t), full(b0),
            full(wih1t), full(whh1t), full(b1),
        ],
        out_specs=[
            pl.BlockSpec((2, B, H), lambda u: (0, 0, 0)),
            pl.BlockSpec((2, B, H), lambda u: (0, 0, 0)),
        ],
        out_shape=[
            jax.ShapeDtypeStruct((2, B, H), jnp.float32),
            jax.ShapeDtypeStruct((2, B, H), jnp.float32),
        ],
        scratch_shapes=[pltpu.VMEM((B, H), jnp.float32) for _ in range(4)],
        compiler_params=pltpu.CompilerParams(
            dimension_semantics=("arbitrary",),
        ),
    )(emb, wih0d, whh0t, b0, wih1t, whh1t, b1)
    return hidden, cell


def kernel(src, emb_table, W_ih0, W_hh0, b_ih0, b_hh0,
           W_ih1, W_hh1, b_ih1, b_hh1):
    B, T = src.shape
    V, E = emb_table.shape
    H = W_hh0.shape[1]
    U = T // 2

    table_pairs = _repack_table(emb_table.T)
    table_view = table_pairs.reshape(2 * table_pairs.shape[0], E)
    emb_wide = _make_gather(B, T, table_view.shape[0], E)(
        src.astype(jnp.int32), table_view)
    emb = emb_wide.reshape(B, U, 1, 2 * E)

    wih0t = W_ih0.T
    z = jnp.zeros((E, 4 * H), jnp.float32)
    wih0d = jnp.block([[wih0t, z], [z, wih0t]])

    hidden, cell = _run_lstm(
        emb,
        wih0d, W_hh0.T, (b_ih0 + b_hh0).reshape(1, 4 * H),
        W_ih1.T, W_hh1.T, (b_ih1 + b_hh1).reshape(1, 4 * H),
    )
    return hidden, cell
